# Initial kernel scaffold; baseline (speedup 1.0000x reference)
import functools

import jax
import jax.numpy as jnp
from jax import lax
from jax.experimental import pallas as pl
from jax.experimental.pallas import tpu as pltpu
from jax.experimental.pallas import tpu_sc as plsc

N = 10000
E = 160000
D = 256
DH = 128
EPS = 1e-5

NC = 2
NS = 16
B = 80
EPW = E // NS
NCHUNK = EPW // B
NPS = N // NS

RB = 1000
NRB = N // RB


def _ln_body(x_ref, g_ref, b_ref, o_ref):
    xb = x_ref[...]
    mu = jnp.mean(xb, axis=-1, keepdims=True)
    xc = xb - mu
    var = jnp.mean(xc * xc, axis=-1, keepdims=True)
    xn = xc * lax.rsqrt(var + EPS) * g_ref[...] + b_ref[...]
    h = jnp.maximum(xn, 0.0)
    o_ref[0] = h[:, :DH]
    o_ref[1] = h[:, DH:]


_ln_call = pl.pallas_call(
    _ln_body,
    grid=(NRB,),
    in_specs=[
        pl.BlockSpec((RB, D), lambda i: (i, 0)),
        pl.BlockSpec((1, D), lambda i: (0, 0)),
        pl.BlockSpec((1, D), lambda i: (0, 0)),
    ],
    out_specs=pl.BlockSpec((2, RB, DH), lambda i: (0, i, 0)),
    out_shape=jax.ShapeDtypeStruct((2, N, DH), jnp.float32),
)


_mesh = plsc.VectorSubcoreMesh(
    core_axis_name="c", subcore_axis_name="s", num_cores=NC, num_subcores=NS
)


@functools.partial(
    pl.kernel,
    out_type=(
        jax.ShapeDtypeStruct((NC * N, DH), jnp.float32),
        jax.ShapeDtypeStruct((N, 16), jnp.float32),
    ),
    mesh=_mesh,
    scratch_types=(
        pltpu.VMEM((B,), jnp.int32),
        pltpu.VMEM((B,), jnp.int32),
        pltpu.VMEM((B, DH), jnp.float32),
        pltpu.VMEM((B, 16), jnp.float32),
        pltpu.VMEM_SHARED((N, DH), jnp.float32),
        pltpu.VMEM_SHARED((N, 16), jnp.float32),
        pltpu.SemaphoreType.DMA,
    ),
)
def _sc_aggregate(src2_hbm, dst_hbm, h2_hbm, zrows_hbm, zcnt_hbm, ones_hbm,
                  agg_out, cnt_out,
                  src_v, dst_v, rows_v, ones_v, acc_sh, cnt_sh, sem):
    c = lax.axis_index("c")
    s = lax.axis_index("s")

    pltpu.sync_copy(zrows_hbm, acc_sh.at[pl.ds(s * NPS, NPS)])

    @pl.when(c == 0)
    def _():
        pltpu.sync_copy(zcnt_hbm, cnt_sh.at[pl.ds(s * NPS, NPS)])

    pltpu.sync_copy(ones_hbm, ones_v)
    plsc.subcore_barrier()

    sbase = c * E + s * EPW
    dbase = s * EPW

    def chunk(j, carry):
        off = sbase + j * B
        doff = dbase + j * B
        pltpu.sync_copy(src2_hbm.at[pl.ds(off, B)], src_v)
        pltpu.sync_copy(dst_hbm.at[pl.ds(doff, B)], dst_v)
        pltpu.async_copy(h2_hbm.at[src_v], rows_v, sem).wait()
        pltpu.sync_copy(rows_v, acc_sh.at[dst_v], add=True)

        @pl.when(c == 0)
        def _():
            pltpu.sync_copy(ones_v, cnt_sh.at[dst_v], add=True)

        return carry

    lax.fori_loop(0, NCHUNK, chunk, 0)
    plsc.subcore_barrier()

    row0 = c * N + s * NPS
    pltpu.sync_copy(acc_sh.at[pl.ds(s * NPS, NPS)],
                    agg_out.at[pl.ds(row0, NPS)])

    @pl.when(c == 0)
    def _():
        pltpu.sync_copy(cnt_sh.at[pl.ds(s * NPS, NPS)],
                        cnt_out.at[pl.ds(s * NPS, NPS)])


def _out_body(a0_ref, a1_ref, c_ref, h0_ref, h1_ref, wl_ref, bl_ref, wr_ref,
              o_ref):
    inv = 1.0 / jnp.maximum(c_ref[:, 0:1], 1.0)
    dn = (((1,), (1,)), ((), ()))
    acc = lax.dot_general(a0_ref[...] * inv, wl_ref[:, :DH], dn,
                          preferred_element_type=jnp.float32)
    acc += lax.dot_general(a1_ref[...] * inv, wl_ref[:, DH:], dn,
                           preferred_element_type=jnp.float32)
    acc += lax.dot_general(h0_ref[...], wr_ref[:, :DH], dn,
                           preferred_element_type=jnp.float32)
    acc += lax.dot_general(h1_ref[...], wr_ref[:, DH:], dn,
                           preferred_element_type=jnp.float32)
    o_ref[...] = acc + bl_ref[...]


_out_call = pl.pallas_call(
    _out_body,
    grid=(NRB,),
    in_specs=[
        pl.BlockSpec((RB, DH), lambda i: (i, 0)),
        pl.BlockSpec((RB, DH), lambda i: (NRB + i, 0)),
        pl.BlockSpec((RB, 16), lambda i: (i, 0)),
        pl.BlockSpec((RB, DH), lambda i: (i, 0)),
        pl.BlockSpec((RB, DH), lambda i: (NRB + i, 0)),
        pl.BlockSpec((D, D), lambda i: (0, 0)),
        pl.BlockSpec((1, D), lambda i: (0, 0)),
        pl.BlockSpec((D, D), lambda i: (0, 0)),
    ],
    out_specs=pl.BlockSpec((RB, D), lambda i: (i, 0)),
    out_shape=jax.ShapeDtypeStruct((N, D), jnp.float32),
)


def kernel(x, edge_index, gamma, beta, W_l, b_l, W_r):
    h2 = _ln_call(x, gamma.reshape(1, D), beta.reshape(1, D))
    h2f = h2.reshape(NC * N, DH)
    src = edge_index[0]
    dst = edge_index[1]
    src2 = jnp.concatenate([src, src + N])
    zrows = jnp.zeros((NPS, DH), jnp.float32)
    zcnt = jnp.zeros((NPS, 16), jnp.float32)
    ones = jnp.ones((B, 16), jnp.float32)
    agg, cnt = _sc_aggregate(src2, dst, h2f, zrows, zcnt, ones)
    return _out_call(agg, agg, cnt, h2f, h2f, W_l, b_l.reshape(1, D), W_r)


# speedup vs baseline: 1.4397x; 1.4397x over previous
import functools

import jax
import jax.numpy as jnp
from jax import lax
from jax.experimental import pallas as pl
from jax.experimental.pallas import tpu as pltpu
from jax.experimental.pallas import tpu_sc as plsc

N = 10000
E = 160000
D = 256
EPS = 1e-5

NPH = 4
DQ = D // NPH

NC = 2
NS = 16
B = 80
EPW = E // NS
NCHUNK = EPW // B
NHALF = N // NC
GROW = NHALF
ACCR = NHALF + 8
SLAB = 312
TAIL = NHALF - NS * SLAB
TAIL_OFF = NS * SLAB

RB = 1000
NRB = N // RB


def _ln_body(x_ref, g_ref, b_ref, o_ref):
    xb = x_ref[...]
    mu = jnp.mean(xb, axis=-1, keepdims=True)
    xc = xb - mu
    var = jnp.mean(xc * xc, axis=-1, keepdims=True)
    xn = xc * lax.rsqrt(var + EPS) * g_ref[...] + b_ref[...]
    h = jnp.maximum(xn, 0.0)
    for q in range(NPH):
        o_ref[q] = h[:, q * DQ:(q + 1) * DQ]


_ln_call = pl.pallas_call(
    _ln_body,
    grid=(NRB,),
    in_specs=[
        pl.BlockSpec((RB, D), lambda i: (i, 0)),
        pl.BlockSpec((1, D), lambda i: (0, 0)),
        pl.BlockSpec((1, D), lambda i: (0, 0)),
    ],
    out_specs=pl.BlockSpec((NPH, RB, DQ), lambda i: (0, i, 0)),
    out_shape=jax.ShapeDtypeStruct((NPH, N, DQ), jnp.float32),
)


_mesh = plsc.VectorSubcoreMesh(
    core_axis_name="c", subcore_axis_name="s", num_cores=NC, num_subcores=NS
)


@functools.partial(
    pl.kernel,
    out_type=(
        jax.ShapeDtypeStruct((NPH * N, DQ), jnp.float32),
        jax.ShapeDtypeStruct((N, 16), jnp.float32),
    ),
    mesh=_mesh,
    compiler_params=pltpu.CompilerParams(use_tc_tiling_on_sc=False),
    scratch_types=(
        pltpu.VMEM((B,), jnp.int32),
        pltpu.VMEM((B,), jnp.int32),
        pltpu.VMEM((B,), jnp.int32),
        pltpu.VMEM((B,), jnp.int32),
        pltpu.VMEM((B, DQ), jnp.float32),
        pltpu.VMEM((B, 16), jnp.float32),
        pltpu.VMEM((SLAB, DQ), jnp.float32),
        pltpu.VMEM((SLAB, 16), jnp.float32),
        pltpu.VMEM_SHARED((ACCR, DQ), jnp.float32),
        pltpu.VMEM_SHARED((ACCR, 16), jnp.float32),
        pltpu.SemaphoreType.DMA,
    ),
)
def _sc_aggregate(src_hbm, dst_hbm, hq_hbm, zrows_hbm, zcnt_hbm, ones_hbm,
                  agg_out, cnt_out,
                  src_v, dst_v, srcp_v, dstl_v, rows_v, ones_v,
                  stage_v, stagec_v, acc_sh, cnt_sh, sem):
    c = lax.axis_index("c")
    s = lax.axis_index("s")
    cbase = c * NHALF

    pltpu.sync_copy(ones_hbm, ones_v)

    for p in range(NPH):
        pltpu.sync_copy(zrows_hbm, stage_v)
        pltpu.sync_copy(stage_v, acc_sh.at[pl.ds(s * SLAB, SLAB)])

        @pl.when(s == 0)
        def _():
            pltpu.sync_copy(stage_v.at[pl.ds(0, TAIL)],
                            acc_sh.at[pl.ds(TAIL_OFF, TAIL)])

        if p == 0:
            pltpu.sync_copy(zcnt_hbm, stagec_v)
            pltpu.sync_copy(stagec_v, cnt_sh.at[pl.ds(s * SLAB, SLAB)])

            @pl.when(s == 0)
            def _():
                pltpu.sync_copy(stagec_v.at[pl.ds(0, TAIL)],
                                cnt_sh.at[pl.ds(TAIL_OFF, TAIL)])

        plsc.subcore_barrier()

        dbase = s * EPW

        def chunk(j, carry):
            doff = dbase + j * B
            pltpu.sync_copy(src_hbm.at[pl.ds(doff, B)], src_v)
            pltpu.sync_copy(dst_hbm.at[pl.ds(doff, B)], dst_v)
            for i in range(B // 16):
                sl = pl.ds(i * 16, 16)
                srcp_v[sl] = src_v[sl] + (p * N)
                t = dst_v[sl] - cbase
                valid = jnp.logical_and(t >= 0, t < NHALF)
                dstl_v[sl] = jnp.where(valid, t, GROW)
            pltpu.async_copy(hq_hbm.at[srcp_v], rows_v, sem).wait()
            pltpu.sync_copy(rows_v, acc_sh.at[dstl_v], add=True)
            if p == 0:
                pltpu.sync_copy(ones_v, cnt_sh.at[dstl_v], add=True)
            return carry

        lax.fori_loop(0, NCHUNK, chunk, 0)
        plsc.subcore_barrier()

        out0 = p * N + cbase
        pltpu.sync_copy(acc_sh.at[pl.ds(s * SLAB, SLAB)], stage_v)
        pltpu.sync_copy(stage_v, agg_out.at[pl.ds(out0 + s * SLAB, SLAB)])

        @pl.when(s == 0)
        def _():
            pltpu.sync_copy(acc_sh.at[pl.ds(TAIL_OFF, TAIL)],
                            stage_v.at[pl.ds(0, TAIL)])
            pltpu.sync_copy(stage_v.at[pl.ds(0, TAIL)],
                            agg_out.at[pl.ds(out0 + TAIL_OFF, TAIL)])

        if p == 0:
            pltpu.sync_copy(cnt_sh.at[pl.ds(s * SLAB, SLAB)], stagec_v)
            pltpu.sync_copy(stagec_v, cnt_out.at[pl.ds(cbase + s * SLAB, SLAB)])

            @pl.when(s == 0)
            def _():
                pltpu.sync_copy(cnt_sh.at[pl.ds(TAIL_OFF, TAIL)],
                                stagec_v.at[pl.ds(0, TAIL)])
                pltpu.sync_copy(stagec_v.at[pl.ds(0, TAIL)],
                                cnt_out.at[pl.ds(cbase + TAIL_OFF, TAIL)])


def _out_body(*refs):
    agg_refs = refs[:NPH]
    c_ref = refs[NPH]
    h_refs = refs[NPH + 1:2 * NPH + 1]
    wl_ref, bl_ref, wr_ref, o_ref = refs[2 * NPH + 1:]
    inv = 1.0 / jnp.maximum(c_ref[:, 0:1], 1.0)
    dn = (((1,), (1,)), ((), ()))
    acc = bl_ref[...] + jnp.zeros((RB, D), jnp.float32)
    for q in range(NPH):
        acc += lax.dot_general(agg_refs[q][...] * inv,
                               wl_ref[:, q * DQ:(q + 1) * DQ], dn,
                               preferred_element_type=jnp.float32)
        acc += lax.dot_general(h_refs[q][...],
                               wr_ref[:, q * DQ:(q + 1) * DQ], dn,
                               preferred_element_type=jnp.float32)
    o_ref[...] = acc


def _group_spec(q):
    return pl.BlockSpec((RB, DQ), lambda i, q=q: (q * NRB + i, 0))


_out_call = pl.pallas_call(
    _out_body,
    grid=(NRB,),
    in_specs=(
        [_group_spec(q) for q in range(NPH)]
        + [pl.BlockSpec((RB, 16), lambda i: (i, 0))]
        + [_group_spec(q) for q in range(NPH)]
        + [
            pl.BlockSpec((D, D), lambda i: (0, 0)),
            pl.BlockSpec((1, D), lambda i: (0, 0)),
            pl.BlockSpec((D, D), lambda i: (0, 0)),
        ]
    ),
    out_specs=pl.BlockSpec((RB, D), lambda i: (i, 0)),
    out_shape=jax.ShapeDtypeStruct((N, D), jnp.float32),
)


def kernel(x, edge_index, gamma, beta, W_l, b_l, W_r):
    hq = _ln_call(x, gamma.reshape(1, D), beta.reshape(1, D))
    hqf = hq.reshape(NPH * N, DQ)
    src = edge_index[0]
    dst = edge_index[1]
    zrows = jnp.zeros((SLAB, DQ), jnp.float32)
    zcnt = jnp.zeros((SLAB, 16), jnp.float32)
    ones = jnp.ones((B, 16), jnp.float32)
    agg, cnt = _sc_aggregate(src, dst, hqf, zrows, zcnt, ones)
    args = ([agg] * NPH) + [cnt] + ([hqf] * NPH)
    return _out_call(*args, W_l, b_l.reshape(1, D), W_r)


# speedup vs baseline: 1.5228x; 1.0577x over previous
import functools

import jax
import jax.numpy as jnp
from jax import lax
from jax.experimental import pallas as pl
from jax.experimental.pallas import tpu as pltpu
from jax.experimental.pallas import tpu_sc as plsc

N = 10000
E = 160000
D = 256
EPS = 1e-5

NPH = 4
DQ = D // NPH

NC = 2
NS = 16
B = 128
NCH = 80
EROWS = NS * NCH
EPAD = EROWS * B - E
NHALF = N // NC
GROW = NHALF
ACCR = NHALF + 8
SLAB = 312
TAIL = NHALF - NS * SLAB
TAIL_OFF = NS * SLAB

RB = 1000
NRB = N // RB


def _ln_body(x_ref, g_ref, b_ref, o_ref):
    xb = x_ref[...]
    mu = jnp.mean(xb, axis=-1, keepdims=True)
    xc = xb - mu
    var = jnp.mean(xc * xc, axis=-1, keepdims=True)
    xn = xc * lax.rsqrt(var + EPS) * g_ref[...] + b_ref[...]
    h = jnp.maximum(xn, 0.0)
    for q in range(NPH):
        o_ref[q] = h[:, q * DQ:(q + 1) * DQ]


_ln_call = pl.pallas_call(
    _ln_body,
    grid=(NRB,),
    in_specs=[
        pl.BlockSpec((RB, D), lambda i: (i, 0)),
        pl.BlockSpec((1, D), lambda i: (0, 0)),
        pl.BlockSpec((1, D), lambda i: (0, 0)),
    ],
    out_specs=pl.BlockSpec((NPH, RB, DQ), lambda i: (0, i, 0)),
    out_shape=jax.ShapeDtypeStruct((NPH, N, DQ), jnp.float32),
)


_mesh = plsc.VectorSubcoreMesh(
    core_axis_name="c", subcore_axis_name="s", num_cores=NC, num_subcores=NS
)


@functools.partial(
    pl.kernel,
    out_type=(
        jax.ShapeDtypeStruct((NPH * N, DQ), jnp.float32),
        jax.ShapeDtypeStruct((N, 16), jnp.float32),
    ),
    mesh=_mesh,
    compiler_params=pltpu.CompilerParams(use_tc_tiling_on_sc=False),
    scratch_types=(
        pltpu.VMEM((NCH, B), jnp.int32),
        pltpu.VMEM((NCH, B), jnp.int32),
        pltpu.VMEM((2, B, DQ), jnp.float32),
        pltpu.VMEM((B, 16), jnp.float32),
        pltpu.VMEM((SLAB, DQ), jnp.float32),
        pltpu.VMEM((SLAB, 16), jnp.float32),
        pltpu.VMEM_SHARED((ACCR, DQ), jnp.float32),
        pltpu.VMEM_SHARED((ACCR, 16), jnp.float32),
        pltpu.SemaphoreType.DMA,
        pltpu.SemaphoreType.DMA,
        pltpu.SemaphoreType.DMA,
        pltpu.SemaphoreType.DMA,
        pltpu.SemaphoreType.DMA,
    ),
)
def _sc_aggregate(src_hbm, dst_hbm, hq_hbm, zrows_hbm, zcnt_hbm, ones_hbm,
                  agg_out, cnt_out,
                  srcp_t, dstl_t, rows_v, ones_v,
                  stage_v, stagec_v, acc_sh, cnt_sh,
                  sem_g0, sem_g1, sem_s0, sem_s1, sem_c):
    c = lax.axis_index("c")
    s = lax.axis_index("s")
    cbase = c * NHALF
    sem_g = (sem_g0, sem_g1)
    sem_s = (sem_s0, sem_s1)

    pltpu.sync_copy(ones_hbm, ones_v)
    pltpu.sync_copy(src_hbm.at[pl.ds(s * NCH, NCH)], srcp_t)
    pltpu.sync_copy(dst_hbm.at[pl.ds(s * NCH, NCH)], dstl_t)

    def _localize(j, carry):
        for i in range(B // 16):
            sl = pl.ds(i * 16, 16)
            t = dstl_t[j, sl] - cbase
            valid = jnp.logical_and(t >= 0, t < NHALF)
            dstl_t[j, sl] = jnp.where(valid, t, GROW)
        return carry

    lax.fori_loop(0, NCH, _localize, 0)

    def _gather(j, d):
        return pltpu.async_copy(hq_hbm.at[srcp_t.at[j]], rows_v.at[d],
                                sem_g[d])

    def _gather_wait(j, d):
        pltpu.make_async_copy(hq_hbm.at[srcp_t.at[j]], rows_v.at[d],
                              sem_g[d]).wait()

    def _scat(j, d):
        pltpu.async_copy(rows_v.at[d], acc_sh.at[dstl_t.at[j]], sem_s[d],
                         add=True)

    def _scat_wait(j, d):
        pltpu.make_async_copy(rows_v.at[d], acc_sh.at[dstl_t.at[j]],
                              sem_s[d]).wait()

    def _cnt(j):
        pltpu.async_copy(ones_v, cnt_sh.at[dstl_t.at[j]], sem_c, add=True)

    def _cnt_wait(j):
        pltpu.make_async_copy(ones_v, cnt_sh.at[dstl_t.at[j]], sem_c).wait()

    for p in range(NPH):
        if p > 0:
            def _bump(j, carry):
                for i in range(B // 16):
                    sl = pl.ds(i * 16, 16)
                    srcp_t[j, sl] = srcp_t[j, sl] + N
                return carry

            lax.fori_loop(0, NCH, _bump, 0)

        pltpu.sync_copy(zrows_hbm, stage_v)
        pltpu.sync_copy(stage_v, acc_sh.at[pl.ds(s * SLAB, SLAB)])

        @pl.when(s == 0)
        def _():
            pltpu.sync_copy(stage_v.at[pl.ds(0, TAIL)],
                            acc_sh.at[pl.ds(TAIL_OFF, TAIL)])

        if p == 0:
            pltpu.sync_copy(zcnt_hbm, stagec_v)
            pltpu.sync_copy(stagec_v, cnt_sh.at[pl.ds(s * SLAB, SLAB)])

            @pl.when(s == 0)
            def _():
                pltpu.sync_copy(stagec_v.at[pl.ds(0, TAIL)],
                                cnt_sh.at[pl.ds(TAIL_OFF, TAIL)])

        plsc.subcore_barrier()

        _gather(0, 0)

        def _pipe(k, carry):
            a = 2 * k
            b = a + 1
            _gather_wait(a, 0)
            _scat(a, 0)

            @pl.when(k > 0)
            def _():
                _scat_wait(a - 1, 1)

            _gather(b, 1)
            if p == 0:
                @pl.when(k > 0)
                def _():
                    _cnt_wait(a - 2)
                    _cnt_wait(a - 1)

                _cnt(a)
                _cnt(b)
            _gather_wait(b, 1)
            _scat(b, 1)

            @pl.when(k < NCH // 2 - 1)
            def _():
                _scat_wait(a, 0)
                _gather(a + 2, 0)

            return carry

        lax.fori_loop(0, NCH // 2, _pipe, 0)
        _scat_wait(NCH - 2, 0)
        _scat_wait(NCH - 1, 1)
        if p == 0:
            _cnt_wait(NCH - 2)
            _cnt_wait(NCH - 1)
        plsc.subcore_barrier()

        out0 = p * N + cbase
        pltpu.sync_copy(acc_sh.at[pl.ds(s * SLAB, SLAB)], stage_v)
        pltpu.sync_copy(stage_v, agg_out.at[pl.ds(out0 + s * SLAB, SLAB)])

        @pl.when(s == 0)
        def _():
            pltpu.sync_copy(acc_sh.at[pl.ds(TAIL_OFF, TAIL)],
                            stage_v.at[pl.ds(0, TAIL)])
            pltpu.sync_copy(stage_v.at[pl.ds(0, TAIL)],
                            agg_out.at[pl.ds(out0 + TAIL_OFF, TAIL)])

        if p == 0:
            pltpu.sync_copy(cnt_sh.at[pl.ds(s * SLAB, SLAB)], stagec_v)
            pltpu.sync_copy(stagec_v, cnt_out.at[pl.ds(cbase + s * SLAB, SLAB)])

            @pl.when(s == 0)
            def _():
                pltpu.sync_copy(cnt_sh.at[pl.ds(TAIL_OFF, TAIL)],
                                stagec_v.at[pl.ds(0, TAIL)])
                pltpu.sync_copy(stagec_v.at[pl.ds(0, TAIL)],
                                cnt_out.at[pl.ds(cbase + TAIL_OFF, TAIL)])


def _out_body(*refs):
    agg_refs = refs[:NPH]
    c_ref = refs[NPH]
    h_refs = refs[NPH + 1:2 * NPH + 1]
    wl_ref, bl_ref, wr_ref, o_ref = refs[2 * NPH + 1:]
    inv = 1.0 / jnp.maximum(c_ref[:, 0:1], 1.0)
    dn = (((1,), (1,)), ((), ()))
    acc = bl_ref[...] + jnp.zeros((RB, D), jnp.float32)
    for q in range(NPH):
        acc += lax.dot_general(agg_refs[q][...] * inv,
                               wl_ref[:, q * DQ:(q + 1) * DQ], dn,
                               preferred_element_type=jnp.float32)
        acc += lax.dot_general(h_refs[q][...],
                               wr_ref[:, q * DQ:(q + 1) * DQ], dn,
                               preferred_element_type=jnp.float32)
    o_ref[...] = acc


def _group_spec(q):
    return pl.BlockSpec((RB, DQ), lambda i, q=q: (q * NRB + i, 0))


_out_call = pl.pallas_call(
    _out_body,
    grid=(NRB,),
    in_specs=(
        [_group_spec(q) for q in range(NPH)]
        + [pl.BlockSpec((RB, 16), lambda i: (i, 0))]
        + [_group_spec(q) for q in range(NPH)]
        + [
            pl.BlockSpec((D, D), lambda i: (0, 0)),
            pl.BlockSpec((1, D), lambda i: (0, 0)),
            pl.BlockSpec((D, D), lambda i: (0, 0)),
        ]
    ),
    out_specs=pl.BlockSpec((RB, D), lambda i: (i, 0)),
    out_shape=jax.ShapeDtypeStruct((N, D), jnp.float32),
)


def kernel(x, edge_index, gamma, beta, W_l, b_l, W_r):
    hq = _ln_call(x, gamma.reshape(1, D), beta.reshape(1, D))
    hqf = hq.reshape(NPH * N, DQ)
    src = jnp.concatenate(
        [edge_index[0], jnp.zeros((EPAD,), jnp.int32)]).reshape(EROWS, B)
    dst = jnp.concatenate(
        [edge_index[1], jnp.full((EPAD,), N, jnp.int32)]).reshape(EROWS, B)
    zrows = jnp.zeros((SLAB, DQ), jnp.float32)
    zcnt = jnp.zeros((SLAB, 16), jnp.float32)
    ones = jnp.ones((B, 16), jnp.float32)
    agg, cnt = _sc_aggregate(src, dst, hqf, zrows, zcnt, ones)
    args = ([agg] * NPH) + [cnt] + ([hqf] * NPH)
    return _out_call(*args, W_l, b_l.reshape(1, D), W_r)


# speedup vs baseline: 2.6967x; 1.7709x over previous
import functools

import jax
import jax.numpy as jnp
from jax import lax
from jax.experimental import pallas as pl
from jax.experimental.pallas import tpu as pltpu
from jax.experimental.pallas import tpu_sc as plsc

N = 10000
E = 160000
D = 256
EPS = 1e-5

NPH = 4
DQ = D // NPH

NC = 2
NS = 16
B = 128
NCH = 80
EROWS = NS * NCH
EPAD = EROWS * B - E
CAP = NCH * B + 2 * B
NHALF = N // NC
GROW = NHALF
ACCR = NHALF + 8
SLAB = 312
TAIL = NHALF - NS * SLAB
TAIL_OFF = NS * SLAB

RB = 1000
NRB = N // RB


def _ln_body(x_ref, g_ref, b_ref, o_ref):
    xb = x_ref[...]
    mu = jnp.mean(xb, axis=-1, keepdims=True)
    xc = xb - mu
    var = jnp.mean(xc * xc, axis=-1, keepdims=True)
    xn = xc * lax.rsqrt(var + EPS) * g_ref[...] + b_ref[...]
    h = jnp.maximum(xn, 0.0)
    for q in range(NPH):
        o_ref[q] = h[:, q * DQ:(q + 1) * DQ]


_ln_call = pl.pallas_call(
    _ln_body,
    grid=(NRB,),
    in_specs=[
        pl.BlockSpec((RB, D), lambda i: (i, 0)),
        pl.BlockSpec((1, D), lambda i: (0, 0)),
        pl.BlockSpec((1, D), lambda i: (0, 0)),
    ],
    out_specs=pl.BlockSpec((NPH, RB, DQ), lambda i: (0, i, 0)),
    out_shape=jax.ShapeDtypeStruct((NPH, N, DQ), jnp.float32),
)


_mesh = plsc.VectorSubcoreMesh(
    core_axis_name="c", subcore_axis_name="s", num_cores=NC, num_subcores=NS
)


@functools.partial(
    pl.kernel,
    out_type=(
        jax.ShapeDtypeStruct((NPH * N, DQ), jnp.float32),
        jax.ShapeDtypeStruct((N, 16), jnp.float32),
    ),
    mesh=_mesh,
    compiler_params=pltpu.CompilerParams(use_tc_tiling_on_sc=False,
                                         needs_layout_passes=False),
    scratch_types=(
        pltpu.VMEM((NCH, B), jnp.int32),
        pltpu.VMEM((NCH, B), jnp.int32),
        pltpu.VMEM((CAP,), jnp.int32),
        pltpu.VMEM((CAP,), jnp.int32),
        pltpu.VMEM((16,), jnp.int32),
        pltpu.VMEM((2, B, DQ), jnp.float32),
        pltpu.VMEM((B, 16), jnp.float32),
        pltpu.VMEM((SLAB, DQ), jnp.float32),
        pltpu.VMEM((SLAB, 16), jnp.float32),
        pltpu.VMEM_SHARED((ACCR, DQ), jnp.float32),
        pltpu.VMEM_SHARED((ACCR, 16), jnp.float32),
        pltpu.SemaphoreType.DMA,
        pltpu.SemaphoreType.DMA,
        pltpu.SemaphoreType.DMA,
        pltpu.SemaphoreType.DMA,
        pltpu.SemaphoreType.DMA,
    ),
)
def _sc_aggregate(src_hbm, dst_hbm, hq_hbm, zrows_hbm, zcnt_hbm, ones_hbm,
                  agg_out, cnt_out,
                  srcr_t, dstr_t, srcc_t, dstc_t, mbuf_v, rows_v, ones_v,
                  stage_v, stagec_v, acc_sh, cnt_sh,
                  sem_g0, sem_g1, sem_s0, sem_s1, sem_c):
    c = lax.axis_index("c")
    s = lax.axis_index("s")
    cbase = c * NHALF
    sem_g = (sem_g0, sem_g1)
    sem_s = (sem_s0, sem_s1)

    pltpu.sync_copy(ones_hbm, ones_v)
    pltpu.sync_copy(src_hbm.at[pl.ds(s * NCH, NCH)], srcr_t)
    pltpu.sync_copy(dst_hbm.at[pl.ds(s * NCH, NCH)], dstr_t)

    def _compact_row(j, mvec):
        for i in range(B // 16):
            sl = pl.ds(i * 16, 16)
            t = dstr_t[j, sl] - cbase
            valid = jnp.logical_and(t >= 0, t < NHALF)
            pos = mvec - 1 + plsc.cumsum(jnp.where(valid, 1, 0))
            plsc.store_scatter(dstc_t, [pos], t, mask=valid)
            plsc.store_scatter(srcc_t, [pos], srcr_t[j, sl], mask=valid)
            mvec = mvec + plsc.all_reduce_population_count(valid)
        return mvec

    mvec = lax.fori_loop(0, NCH, _compact_row,
                         jnp.zeros((16,), jnp.int32))
    mbuf_v[...] = mvec
    m = mbuf_v[pl.ds(0, 16)][0]

    mpad = ((m + 2 * B - 1) // (2 * B)) * (2 * B)
    zeros16 = jnp.zeros((16,), jnp.int32)
    grow16 = zeros16 + GROW

    def _pad(g, carry):
        idx = m + g * 16 + lax.iota(jnp.int32, 16)
        mask = idx < mpad
        plsc.store_scatter(dstc_t, [idx], grow16, mask=mask)
        plsc.store_scatter(srcc_t, [idx], zeros16, mask=mask)
        return carry

    lax.fori_loop(0, 2 * B // 16, _pad, 0)
    npair = mpad // (2 * B)

    def _gather(j, d):
        return pltpu.async_copy(hq_hbm.at[srcc_t.at[pl.ds(j * B, B)]],
                                rows_v.at[d], sem_g[d])

    def _gather_wait(j, d):
        pltpu.make_async_copy(hq_hbm.at[srcc_t.at[pl.ds(j * B, B)]],
                              rows_v.at[d], sem_g[d]).wait()

    def _scat(j, d):
        pltpu.async_copy(rows_v.at[d], acc_sh.at[dstc_t.at[pl.ds(j * B, B)]],
                         sem_s[d], add=True)

    def _scat_wait(j, d):
        pltpu.make_async_copy(rows_v.at[d],
                              acc_sh.at[dstc_t.at[pl.ds(j * B, B)]],
                              sem_s[d]).wait()

    def _cnt(j):
        pltpu.async_copy(ones_v, cnt_sh.at[dstc_t.at[pl.ds(j * B, B)]],
                         sem_c, add=True)

    def _cnt_wait(j):
        pltpu.make_async_copy(ones_v, cnt_sh.at[dstc_t.at[pl.ds(j * B, B)]],
                              sem_c).wait()

    for p in range(NPH):
        if p > 0:
            def _bump(g, carry):
                sl = pl.ds(g * 16, 16)
                srcc_t[sl] = srcc_t[sl] + N
                return carry

            lax.fori_loop(0, CAP // 16, _bump, 0)

        pltpu.sync_copy(zrows_hbm, stage_v)
        pltpu.sync_copy(stage_v, acc_sh.at[pl.ds(s * SLAB, SLAB)])

        @pl.when(s == 0)
        def _():
            pltpu.sync_copy(stage_v.at[pl.ds(0, TAIL)],
                            acc_sh.at[pl.ds(TAIL_OFF, TAIL)])

        if p == 0:
            pltpu.sync_copy(zcnt_hbm, stagec_v)
            pltpu.sync_copy(stagec_v, cnt_sh.at[pl.ds(s * SLAB, SLAB)])

            @pl.when(s == 0)
            def _():
                pltpu.sync_copy(stagec_v.at[pl.ds(0, TAIL)],
                                cnt_sh.at[pl.ds(TAIL_OFF, TAIL)])

        plsc.subcore_barrier()

        @pl.when(npair > 0)
        def _():
            _gather(0, 0)

        def _pipe(k, carry):
            a = 2 * k
            b = a + 1
            _gather_wait(a, 0)
            _scat(a, 0)

            @pl.when(k > 0)
            def _():
                _scat_wait(a - 1, 1)

            _gather(b, 1)
            if p == 0:
                @pl.when(k > 0)
                def _():
                    _cnt_wait(a - 2)
                    _cnt_wait(a - 1)

                _cnt(a)
                _cnt(b)
            _gather_wait(b, 1)
            _scat(b, 1)

            @pl.when(k < npair - 1)
            def _():
                _scat_wait(a, 0)
                _gather(a + 2, 0)

            return carry

        lax.fori_loop(0, npair, _pipe, 0)

        @pl.when(npair > 0)
        def _():
            _scat_wait(2 * npair - 2, 0)
            _scat_wait(2 * npair - 1, 1)
            if p == 0:
                _cnt_wait(2 * npair - 2)
                _cnt_wait(2 * npair - 1)

        plsc.subcore_barrier()

        out0 = p * N + cbase
        pltpu.sync_copy(acc_sh.at[pl.ds(s * SLAB, SLAB)], stage_v)
        pltpu.sync_copy(stage_v, agg_out.at[pl.ds(out0 + s * SLAB, SLAB)])

        @pl.when(s == 0)
        def _():
            pltpu.sync_copy(acc_sh.at[pl.ds(TAIL_OFF, TAIL)],
                            stage_v.at[pl.ds(0, TAIL)])
            pltpu.sync_copy(stage_v.at[pl.ds(0, TAIL)],
                            agg_out.at[pl.ds(out0 + TAIL_OFF, TAIL)])

        if p == 0:
            pltpu.sync_copy(cnt_sh.at[pl.ds(s * SLAB, SLAB)], stagec_v)
            pltpu.sync_copy(stagec_v, cnt_out.at[pl.ds(cbase + s * SLAB, SLAB)])

            @pl.when(s == 0)
            def _():
                pltpu.sync_copy(cnt_sh.at[pl.ds(TAIL_OFF, TAIL)],
                                stagec_v.at[pl.ds(0, TAIL)])
                pltpu.sync_copy(stagec_v.at[pl.ds(0, TAIL)],
                                cnt_out.at[pl.ds(cbase + TAIL_OFF, TAIL)])


def _out_body(*refs):
    agg_refs = refs[:NPH]
    c_ref = refs[NPH]
    h_refs = refs[NPH + 1:2 * NPH + 1]
    wl_ref, bl_ref, wr_ref, o_ref = refs[2 * NPH + 1:]
    inv = 1.0 / jnp.maximum(c_ref[:, 0:1], 1.0)
    dn = (((1,), (1,)), ((), ()))
    acc = bl_ref[...] + jnp.zeros((RB, D), jnp.float32)
    for q in range(NPH):
        acc += lax.dot_general(agg_refs[q][...] * inv,
                               wl_ref[:, q * DQ:(q + 1) * DQ], dn,
                               preferred_element_type=jnp.float32)
        acc += lax.dot_general(h_refs[q][...],
                               wr_ref[:, q * DQ:(q + 1) * DQ], dn,
                               preferred_element_type=jnp.float32)
    o_ref[...] = acc


def _group_spec(q):
    return pl.BlockSpec((RB, DQ), lambda i, q=q: (q * NRB + i, 0))


_out_call = pl.pallas_call(
    _out_body,
    grid=(NRB,),
    in_specs=(
        [_group_spec(q) for q in range(NPH)]
        + [pl.BlockSpec((RB, 16), lambda i: (i, 0))]
        + [_group_spec(q) for q in range(NPH)]
        + [
            pl.BlockSpec((D, D), lambda i: (0, 0)),
            pl.BlockSpec((1, D), lambda i: (0, 0)),
            pl.BlockSpec((D, D), lambda i: (0, 0)),
        ]
    ),
    out_specs=pl.BlockSpec((RB, D), lambda i: (i, 0)),
    out_shape=jax.ShapeDtypeStruct((N, D), jnp.float32),
)


def kernel(x, edge_index, gamma, beta, W_l, b_l, W_r):
    hq = _ln_call(x, gamma.reshape(1, D), beta.reshape(1, D))
    hqf = hq.reshape(NPH * N, DQ)
    src = jnp.concatenate(
        [edge_index[0], jnp.zeros((EPAD,), jnp.int32)]).reshape(EROWS, B)
    dst = jnp.concatenate(
        [edge_index[1], jnp.full((EPAD,), N, jnp.int32)]).reshape(EROWS, B)
    zrows = jnp.zeros((SLAB, DQ), jnp.float32)
    zcnt = jnp.zeros((SLAB, 16), jnp.float32)
    ones = jnp.ones((B, 16), jnp.float32)
    agg, cnt = _sc_aggregate(src, dst, hqf, zrows, zcnt, ones)
    args = ([agg] * NPH) + [cnt] + ([hqf] * NPH)
    return _out_call(*args, W_l, b_l.reshape(1, D), W_r)
